# baseline (speedup 1.0000x reference)
import functools

import jax
import jax.numpy as jnp
from jax import lax
from jax.experimental import pallas as pl
from jax.experimental.pallas import tpu as pltpu
from jax.experimental.pallas import tpu_sc as plsc

_NW = 32
_LANES = 16
_ROWS = 8
_COLS = 512
_NBUF = 2
_INBUF = 4


def _sc_call(num_heads, num_dist, n):
    rows_per_w = n // _NW
    nchunk = (rows_per_w // _ROWS) * (n // _COLS)
    col_groups = n // _COLS
    mesh = plsc.VectorSubcoreMesh(core_axis_name="c", subcore_axis_name="s")

    @functools.partial(
        pl.kernel,
        mesh=mesh,
        out_type=jax.ShapeDtypeStruct((num_heads, n, n), jnp.float32),
        compiler_params=pltpu.CompilerParams(
            needs_layout_passes=False, use_tc_tiling_on_sc=True
        ),
        scratch_types=[
            pltpu.VMEM((num_heads, 128), jnp.float32),
            pltpu.VMEM((num_heads, 128), jnp.float32),
            pltpu.VMEM((num_heads, 128), jnp.float32),
            pltpu.VMEM((_INBUF, _ROWS, _COLS), jnp.int32),
            pltpu.VMEM((_INBUF, _ROWS, _COLS), jnp.int32),
            pltpu.VMEM((_NBUF, num_heads, _ROWS, _COLS), jnp.float32),
            pltpu.SemaphoreType.DMA,
            pltpu.SemaphoreType.DMA,
        ],
    )
    def body(d_hbm, e_hbm, dw_hbm, ew_hbm, out_hbm, dw_v, ew_v, tab_v, d_v,
             e_v, o_v, sem_in, sem_out):
        wid = lax.axis_index("s") * 2 + lax.axis_index("c")
        row_base = wid * rows_per_w
        pltpu.sync_copy(dw_hbm, dw_v)
        pltpu.sync_copy(ew_hbm, ew_v)
        for h in range(num_heads):
            ew_row = ew_v[h, pl.ds(0, _LANES)]
            dw_row = dw_v[h, pl.ds(0, _LANES)]
            for d in range(num_dist):
                tab_v[h, pl.ds(d * _LANES, _LANES)] = ew_row + dw_row[d]

        def chunk_origin(g):
            t = g // col_groups
            q = g % col_groups
            return row_base + t * _ROWS, q * _COLS

        def issue_in(g, b):
            r0, c0 = chunk_origin(g)
            pltpu.async_copy(
                d_hbm.at[pl.ds(r0, _ROWS), pl.ds(c0, _COLS)], d_v.at[b], sem_in
            )
            pltpu.async_copy(
                e_hbm.at[pl.ds(r0, _ROWS), pl.ds(c0, _COLS)], e_v.at[b], sem_in
            )

        def wait_in(b):
            pltpu.make_async_copy(
                d_hbm.at[pl.ds(0, _ROWS), pl.ds(0, _COLS)], d_v.at[b], sem_in
            ).wait()
            pltpu.make_async_copy(
                e_hbm.at[pl.ds(0, _ROWS), pl.ds(0, _COLS)], e_v.at[b], sem_in
            ).wait()

        def issue_out(g, b):
            r0, c0 = chunk_origin(g)
            for h in range(num_heads):
                pltpu.async_copy(
                    o_v.at[b, h],
                    out_hbm.at[h, pl.ds(r0, _ROWS), pl.ds(c0, _COLS)],
                    sem_out,
                )

        def wait_out(b):
            for h in range(num_heads):
                pltpu.make_async_copy(
                    o_v.at[b, h],
                    out_hbm.at[0, pl.ds(0, _ROWS), pl.ds(0, _COLS)],
                    sem_out,
                ).wait()

        for b in range(2):
            issue_in(b, b)

        @pl.loop(0, nchunk, step=_INBUF)
        def _(g0):
            for b in range(_INBUF):
                g = g0 + b
                ob = b % _NBUF

                @pl.when(g + 2 < nchunk)
                def _():
                    issue_in(g + 2, (b + 2) % _INBUF)

                wait_in(b)

                @pl.when(g >= _NBUF)
                def _():
                    wait_out(ob)

                for r in range(_ROWS):
                    @plsc.parallel_loop(0, _COLS // _LANES, unroll=16)
                    def _(i):
                        dvec = d_v[b, r, pl.ds(i * _LANES, _LANES)]
                        evec = e_v[b, r, pl.ds(i * _LANES, _LANES)]
                        cidx = (dvec << 4) | evec
                        for h in range(num_heads):
                            o_v[ob, h, r, pl.ds(i * _LANES, _LANES)] = (
                                plsc.load_gather(tab_v.at[h], [cidx])
                            )

                issue_out(g, ob)

        for b in range(_NBUF):
            wait_out(b)

    return body


def kernel(dist_matrix, edge_type_matrix, dist_bias_weight, edge_type_bias_weight):
    n = dist_matrix.shape[0]
    num_dist, num_heads = dist_bias_weight.shape
    dw_t = jnp.zeros((num_heads, 128), jnp.float32).at[:, :num_dist].set(
        dist_bias_weight.T
    )
    ew_t = jnp.zeros((num_heads, 128), jnp.float32).at[:, : edge_type_bias_weight.shape[0]].set(
        edge_type_bias_weight.T.astype(jnp.float32)
    )
    return _sc_call(num_heads, num_dist, n)(
        dist_matrix, edge_type_matrix, dw_t, ew_t
    )


# speedup vs baseline: 1.0789x; 1.0789x over previous
import functools

import jax
import jax.numpy as jnp
from jax import lax
from jax.experimental import pallas as pl
from jax.experimental.pallas import tpu as pltpu
from jax.experimental.pallas import tpu_sc as plsc

_NW = 32
_LANES = 16
_ROWS = 8
_COLS = 512
_NBUF = 2


def _sc_call(num_heads, num_dist, n):
    rows_per_w = n // _NW
    nchunk = (rows_per_w // _ROWS) * (n // _COLS)
    col_groups = n // _COLS
    mesh = plsc.VectorSubcoreMesh(core_axis_name="c", subcore_axis_name="s")

    @functools.partial(
        pl.kernel,
        mesh=mesh,
        out_type=jax.ShapeDtypeStruct((num_heads, n, n), jnp.float32),
        compiler_params=pltpu.CompilerParams(
            needs_layout_passes=False, use_tc_tiling_on_sc=True
        ),
        scratch_types=[
            pltpu.VMEM((num_heads, 128), jnp.float32),
            pltpu.VMEM((num_heads, 128), jnp.float32),
            pltpu.VMEM((num_heads, 128), jnp.float32),
            pltpu.VMEM((_NBUF, _ROWS, _COLS), jnp.int32),
            pltpu.VMEM((_NBUF, _ROWS, _COLS), jnp.int32),
            pltpu.VMEM((_NBUF, num_heads, _ROWS, _COLS), jnp.float32),
            pltpu.SemaphoreType.DMA,
            pltpu.SemaphoreType.DMA,
        ],
    )
    def body(d_hbm, e_hbm, dw_hbm, ew_hbm, out_hbm, dw_v, ew_v, tab_v, d_v,
             e_v, o_v, sem_in, sem_out):
        wid = lax.axis_index("s") * 2 + lax.axis_index("c")
        row_base = wid * rows_per_w
        pltpu.sync_copy(dw_hbm, dw_v)
        pltpu.sync_copy(ew_hbm, ew_v)
        for h in range(num_heads):
            ew_row = ew_v[h, pl.ds(0, _LANES)]
            dw_row = dw_v[h, pl.ds(0, _LANES)]
            for d in range(num_dist):
                tab_v[h, pl.ds(d * _LANES, _LANES)] = ew_row + dw_row[d]

        def chunk_origin(g):
            t = g // col_groups
            q = g % col_groups
            return row_base + t * _ROWS, q * _COLS

        def issue_in(g, b):
            r0, c0 = chunk_origin(g)
            pltpu.async_copy(
                d_hbm.at[pl.ds(r0, _ROWS), pl.ds(c0, _COLS)], d_v.at[b], sem_in
            )
            pltpu.async_copy(
                e_hbm.at[pl.ds(r0, _ROWS), pl.ds(c0, _COLS)], e_v.at[b], sem_in
            )

        def wait_in(b):
            pltpu.make_async_copy(
                d_hbm.at[pl.ds(0, _ROWS), pl.ds(0, _COLS)], d_v.at[b], sem_in
            ).wait()
            pltpu.make_async_copy(
                e_hbm.at[pl.ds(0, _ROWS), pl.ds(0, _COLS)], e_v.at[b], sem_in
            ).wait()

        def issue_out(g, b):
            r0, c0 = chunk_origin(g)
            for h in range(num_heads):
                pltpu.async_copy(
                    o_v.at[b, h],
                    out_hbm.at[h, pl.ds(r0, _ROWS), pl.ds(c0, _COLS)],
                    sem_out,
                )

        def wait_out(b):
            for h in range(num_heads):
                pltpu.make_async_copy(
                    o_v.at[b, h],
                    out_hbm.at[0, pl.ds(0, _ROWS), pl.ds(0, _COLS)],
                    sem_out,
                ).wait()

        for b in range(_NBUF):
            issue_in(b, b)

        @pl.loop(0, nchunk, step=_NBUF)
        def _(g0):
            for b in range(_NBUF):
                g = g0 + b
                wait_in(b)

                @pl.when(g >= _NBUF)
                def _():
                    wait_out(b)

                for r in range(_ROWS):
                    @plsc.parallel_loop(0, _COLS // _LANES, unroll=16)
                    def _(i):
                        dvec = d_v[b, r, pl.ds(i * _LANES, _LANES)]
                        evec = e_v[b, r, pl.ds(i * _LANES, _LANES)]
                        cidx = (dvec << 4) | evec
                        for h in range(num_heads):
                            o_v[b, h, r, pl.ds(i * _LANES, _LANES)] = (
                                plsc.load_gather(tab_v.at[h], [cidx])
                            )

                issue_out(g, b)

                @pl.when(g + _NBUF < nchunk)
                def _():
                    issue_in(g + _NBUF, b)

        for b in range(_NBUF):
            wait_out(b)

    return body


def kernel(dist_matrix, edge_type_matrix, dist_bias_weight, edge_type_bias_weight):
    n = dist_matrix.shape[0]
    num_dist, num_heads = dist_bias_weight.shape
    dw_t = jnp.zeros((num_heads, 128), jnp.float32).at[:, :num_dist].set(
        dist_bias_weight.T
    )
    ew_t = jnp.zeros((num_heads, 128), jnp.float32).at[:, : edge_type_bias_weight.shape[0]].set(
        edge_type_bias_weight.T.astype(jnp.float32)
    )
    return _sc_call(num_heads, num_dist, n)(
        dist_matrix, edge_type_matrix, dw_t, ew_t
    )


# speedup vs baseline: 1.1026x; 1.0219x over previous
import functools

import jax
import jax.numpy as jnp
from jax import lax
from jax.experimental import pallas as pl
from jax.experimental.pallas import tpu as pltpu
from jax.experimental.pallas import tpu_sc as plsc

_NW = 32
_LANES = 16
_ROWS = 8
_COLS = 512
_NBUF = 2


def _sc_call(num_heads, num_dist, n):
    rows_per_w = n // _NW
    nchunk = (rows_per_w // _ROWS) * (n // _COLS)
    col_groups = n // _COLS
    mesh = plsc.VectorSubcoreMesh(core_axis_name="c", subcore_axis_name="s")

    @functools.partial(
        pl.kernel,
        mesh=mesh,
        out_type=jax.ShapeDtypeStruct((num_heads, n, n), jnp.float32),
        compiler_params=pltpu.CompilerParams(
            needs_layout_passes=False, use_tc_tiling_on_sc=True
        ),
        scratch_types=[
            pltpu.VMEM((num_heads, 128), jnp.float32),
            pltpu.VMEM((num_heads, 128), jnp.float32),
            pltpu.VMEM((num_heads, 128), jnp.float32),
            pltpu.VMEM((_NBUF, _ROWS, _COLS), jnp.int32),
            pltpu.VMEM((_NBUF, _ROWS, _COLS), jnp.int32),
            pltpu.VMEM((_NBUF, num_heads, _ROWS, _COLS), jnp.float32),
            pltpu.SemaphoreType.DMA,
            pltpu.SemaphoreType.DMA,
        ],
    )
    def body(d_hbm, e_hbm, dw_hbm, ew_hbm, out_hbm, dw_v, ew_v, tab_v, d_v,
             e_v, o_v, sem_in, sem_out):
        wid = lax.axis_index("s") * 2 + lax.axis_index("c")
        row_base = wid * rows_per_w

        def chunk_origin(g):
            t = g // col_groups
            q = g % col_groups
            return row_base + t * _ROWS, q * _COLS

        def issue_in(g, b):
            r0, c0 = chunk_origin(g)
            pltpu.async_copy(
                d_hbm.at[pl.ds(r0, _ROWS), pl.ds(c0, _COLS)], d_v.at[b], sem_in
            )
            pltpu.async_copy(
                e_hbm.at[pl.ds(r0, _ROWS), pl.ds(c0, _COLS)], e_v.at[b], sem_in
            )

        def wait_in(b):
            pltpu.make_async_copy(
                d_hbm.at[pl.ds(0, _ROWS), pl.ds(0, _COLS)], d_v.at[b], sem_in
            ).wait()
            pltpu.make_async_copy(
                e_hbm.at[pl.ds(0, _ROWS), pl.ds(0, _COLS)], e_v.at[b], sem_in
            ).wait()

        def issue_out(g, b):
            r0, c0 = chunk_origin(g)
            for h in range(num_heads):
                pltpu.async_copy(
                    o_v.at[b, h],
                    out_hbm.at[h, pl.ds(r0, _ROWS), pl.ds(c0, _COLS)],
                    sem_out,
                )

        def wait_out(b):
            for h in range(num_heads):
                pltpu.make_async_copy(
                    o_v.at[b, h],
                    out_hbm.at[0, pl.ds(0, _ROWS), pl.ds(0, _COLS)],
                    sem_out,
                ).wait()

        for b in range(_NBUF):
            issue_in(b, b)
        w1 = pltpu.async_copy(dw_hbm, dw_v, sem_out)
        w2 = pltpu.async_copy(ew_hbm, ew_v, sem_out)
        w1.wait()
        w2.wait()
        for h in range(num_heads):
            ew_row = ew_v[h, pl.ds(0, _LANES)]
            dw_row = dw_v[h, pl.ds(0, _LANES)]
            for d in range(num_dist):
                tab_v[h, pl.ds(d * _LANES, _LANES)] = ew_row + dw_row[d]

        @pl.loop(0, nchunk, step=_NBUF)
        def _(g0):
            for b in range(_NBUF):
                g = g0 + b
                wait_in(b)

                @pl.when(g >= _NBUF)
                def _():
                    wait_out(b)

                for r in range(_ROWS):
                    @plsc.parallel_loop(0, _COLS // _LANES, unroll=16)
                    def _(i):
                        dvec = d_v[b, r, pl.ds(i * _LANES, _LANES)]
                        evec = e_v[b, r, pl.ds(i * _LANES, _LANES)]
                        cidx = (dvec << 4) | evec
                        for h in range(num_heads):
                            o_v[b, h, r, pl.ds(i * _LANES, _LANES)] = (
                                plsc.load_gather(tab_v.at[h], [cidx])
                            )

                issue_out(g, b)

                @pl.when(g + _NBUF < nchunk)
                def _():
                    issue_in(g + _NBUF, b)

        for b in range(_NBUF):
            wait_out(b)

    return body


def kernel(dist_matrix, edge_type_matrix, dist_bias_weight, edge_type_bias_weight):
    n = dist_matrix.shape[0]
    num_dist, num_heads = dist_bias_weight.shape
    dw_t = jnp.zeros((num_heads, 128), jnp.float32).at[:, :num_dist].set(
        dist_bias_weight.T
    )
    ew_t = jnp.zeros((num_heads, 128), jnp.float32).at[:, : edge_type_bias_weight.shape[0]].set(
        edge_type_bias_weight.T.astype(jnp.float32)
    )
    return _sc_call(num_heads, num_dist, n)(
        dist_matrix, edge_type_matrix, dw_t, ew_t
    )
